# Initial kernel scaffold; baseline (speedup 1.0000x reference)
import jax
import jax.numpy as jnp
from jax.experimental import pallas as pl


def kernel(patch_tokens, anomaly_maps):
    raise NotImplementedError("write your pallas kernel here")


# speedup vs baseline: 1.5521x; 1.5521x over previous
import jax
import jax.numpy as jnp
from jax.experimental import pallas as pl


def _tc_body(am_ref, pt_ref, out_ref):
    a = am_ref[:, 0]
    d = a[:, 1, :] - a[:, 0, :]
    d = jnp.sum(d, axis=0, keepdims=True) * 0.25
    w = jax.nn.sigmoid(d)

    L = pt_ref.shape[0]
    acc = jnp.zeros((1, pt_ref.shape[3]), dtype=jnp.float32)
    for l in range(L):
        acc = acc + jnp.dot(w, pt_ref[l, 0], preferred_element_type=jnp.float32)
    s = acc * (1.0 / L)

    n = jnp.sqrt(jnp.sum(s * s, axis=1, keepdims=True))
    out_ref[...] = (s / jnp.maximum(n, 1e-12))[:, None, :]


def kernel(patch_tokens, anomaly_maps):
    L, B, T, D = patch_tokens.shape
    am_t = jnp.swapaxes(anomaly_maps, 2, 3)

    out = pl.pallas_call(
        _tc_body,
        grid=(B,),
        in_specs=[
            pl.BlockSpec((L, 1, 2, T), lambda b: (0, b, 0, 0)),
            pl.BlockSpec((L, 1, T, D), lambda b: (0, b, 0, 0)),
        ],
        out_specs=pl.BlockSpec((1, 1, D), lambda b: (b, 0, 0)),
        out_shape=jax.ShapeDtypeStruct((B, 1, D), jnp.float32),
    )(am_t, patch_tokens)
    return out.reshape(B, D)
